# Initial kernel scaffold; baseline (speedup 1.0000x reference)
import jax
import jax.numpy as jnp
from jax.experimental import pallas as pl


def kernel(x, edge_index, batch, W1, b1, Ws1, bs1, W2, b2, Ws2, bs2, W3, b3, Ws3, bs3, Wl1, bl1, Wl2, bl2, Wl3, bl3):
    raise NotImplementedError("write your pallas kernel here")


# speedup vs baseline: 84.0414x; 84.0414x over previous
import functools

import jax
import jax.numpy as jnp
from jax import lax
from jax.experimental import pallas as pl
from jax.experimental.pallas import tpu as pltpu
from jax.experimental.pallas import tpu_sc as plsc

_N, _E, _G, _NPER = 10000, 320000, 50, 200
_PAD = 256
_NCLS = 10

_NSUB = 16
_EPT = _E // _NSUB
_ECH = 2000
_NCH = _EPT // _ECH
_ROWS = (_ECH + 127) // 128
_SLOT = _ROWS * 128
_HALF = _N // 2
_HALFE = _HALF * _PAD
_STRIPE = _HALFE // _NSUB
_ZCH = 8000


def _build_adj(edge_index):
    mesh = plsc.VectorSubcoreMesh(core_axis_name="c", subcore_axis_name="s")

    @functools.partial(
        pl.kernel,
        mesh=mesh,
        out_type=jax.ShapeDtypeStruct((_N * _PAD,), jnp.float32),
        scratch_types=[
            pltpu.VMEM((_SLOT,), jnp.int32),
            pltpu.VMEM((_SLOT,), jnp.int32),
            pltpu.VMEM((_ROWS, 128), jnp.int32),
            pltpu.VMEM((128,), jnp.float32),
            pltpu.VMEM((_ZCH,), jnp.float32),
            pltpu.VMEM_SHARED((_HALFE,), jnp.float32),
        ],
    )
    def build(src_hbm, dst_hbm, out_hbm, src_v, dst_v, idx_v, ones_v, z_v, shared):
        c = lax.axis_index("c")
        s = lax.axis_index("s")

        def fill_ones(i, _):
            ones_v[pl.ds(i * 16, 16)] = jnp.ones((16,), jnp.float32)
            return 0

        lax.fori_loop(0, 128 // 16, fill_ones, 0)

        def fill_z(i, _):
            z_v[pl.ds(i * 16, 16)] = jnp.zeros((16,), jnp.float32)
            return 0

        lax.fori_loop(0, _ZCH // 16, fill_z, 0)

        def zstripe(q, _):
            pltpu.sync_copy(z_v, shared.at[pl.ds(s * _STRIPE + q * _ZCH, _ZCH)])
            return 0

        lax.fori_loop(0, _STRIPE // _ZCH, zstripe, 0)

        plsc.subcore_barrier()
        lo = c * _HALF

        def chunk(q, _):
            off = s * _EPT + q * _ECH
            pltpu.sync_copy(src_hbm.at[pl.ds(off, _ECH)],
                            src_v.at[pl.ds(0, _ECH)])
            pltpu.sync_copy(dst_hbm.at[pl.ds(off, _ECH)],
                            dst_v.at[pl.ds(0, _ECH)])

            def rowfn(j, _):
                for v in range(8):
                    base = j * 128 + v * 16
                    sv = src_v[pl.ds(base, 16)]
                    dv = dst_v[pl.ds(base, 16)]
                    col = lax.rem(sv, 200)
                    dl = dv - lo
                    lane = base + lax.iota(jnp.int32, 16)
                    ok = (dl >= 0) & (dl < _HALF) & (lane < _ECH)
                    idx_v[j, pl.ds(v * 16, 16)] = jnp.where(
                        ok, dl * _PAD + col, 255)
                pltpu.sync_copy(ones_v, shared.at[idx_v.at[j]], add=True)
                return 0

            lax.fori_loop(0, _ROWS, rowfn, 0)
            return 0

        lax.fori_loop(0, _NCH, chunk, 0)
        plsc.subcore_barrier()
        pltpu.sync_copy(shared.at[pl.ds(s * _STRIPE, _STRIPE)],
                        out_hbm.at[pl.ds(c * _HALFE + s * _STRIPE, _STRIPE)])

    return build(edge_index[0], edge_index[1])


def _dinv_t(A, active):
    ap = jnp.concatenate(
        [active, jnp.zeros((_PAD - _NPER, 1), jnp.float32)], axis=0)
    deg = jnp.dot(A, ap, preferred_element_type=jnp.float32) * active + active
    return jnp.where(deg > 0, 1.0 / jnp.sqrt(jnp.maximum(deg, 1e-12)), 0.0)


def _gcn_t(A, h, W, b, active, dinv):
    hw = jnp.dot(h, W, preferred_element_type=jnp.float32)
    hp = jnp.concatenate(
        [dinv * hw, jnp.zeros((_PAD - _NPER, hw.shape[1]), jnp.float32)], axis=0)
    agg = jnp.dot(A, hp, preferred_element_type=jnp.float32) * dinv
    return (agg + (dinv * dinv * active) * hw + b) * active


def _layer_t(A, h_in, active, W, b, Ws, bs, k, eye, ilt):
    dinv = _dinv_t(A, active)
    h = jax.nn.relu(_gcn_t(A, h_in, W, b, active, dinv))
    sc = _gcn_t(A, h, Ws, bs, active, dinv)
    sm = jnp.where(active > 0, sc, -1e9)
    smT = lax.dot_general(sm, eye, (((0,), (0,)), ((), ())))
    beats = (smT > sm) | ((smT == sm) & ilt)
    cnt = jnp.sum(jnp.where(beats, 1.0, 0.0), axis=1, keepdims=True)
    na = jnp.where(cnt < k, 1.0, 0.0) * active
    hn = h * jnp.tanh(sc) * na
    xmax = jnp.max(jnp.where(na > 0, hn, -1e9), axis=0, keepdims=True)
    cntn = jnp.maximum(jnp.sum(na), 1.0)
    xmean = jnp.sum(hn, axis=0, keepdims=True) / cntn
    return hn, na, jnp.concatenate([xmax, xmean], axis=1)


def _fwd_body(a_ref, x_ref, W1r, b1r, Ws1r, bs1r, W2r, b2r, Ws2r, bs2r,
              W3r, b3r, Ws3r, bs3r, Wl1r, bl1r, Wl2r, bl2r, Wl3r, bl3r,
              o_ref):
    f32 = jnp.float32
    A = a_ref[0]
    x = x_ref[0]
    ii = lax.broadcasted_iota(jnp.int32, (_NPER, _NPER), 0)
    jj = lax.broadcasted_iota(jnp.int32, (_NPER, _NPER), 1)
    eye = jnp.where(ii == jj, 1.0, 0.0).astype(f32)
    ilt = jj < ii
    active = jnp.ones((_NPER, 1), f32)
    h, active, T1 = _layer_t(A, x, active, W1r[...], b1r[...], Ws1r[...],
                             bs1r[...], 100, eye, ilt)
    h, active, T2 = _layer_t(A, h, active, W2r[...], b2r[...], Ws2r[...],
                             bs2r[...], 50, eye, ilt)
    h, active, T3 = _layer_t(A, h, active, W3r[...], b3r[...], Ws3r[...],
                             bs3r[...], 25, eye, ilt)
    z = jnp.concatenate([T1, T2, T3], axis=1)
    z = jax.nn.relu(jnp.dot(z, Wl1r[...], preferred_element_type=f32) + bl1r[...])
    z = jax.nn.relu(jnp.dot(z, Wl2r[...], preferred_element_type=f32) + bl2r[...])
    z = jnp.dot(z, Wl3r[...], preferred_element_type=f32) + bl3r[...]
    m = jnp.max(z, axis=1, keepdims=True)
    o_ref[0] = (z - m) - jnp.log(jnp.sum(jnp.exp(z - m), axis=1, keepdims=True))


def _forward_tc(A3, x3, *weights):
    def fullspec(arr):
        nd = arr.ndim
        return pl.BlockSpec(arr.shape, lambda g, _nd=nd: (0,) * _nd)

    in_specs = [
        pl.BlockSpec((1, _NPER, _PAD), lambda g: (g, 0, 0)),
        pl.BlockSpec((1, _NPER, 128), lambda g: (g, 0, 0)),
    ] + [fullspec(w) for w in weights]
    return pl.pallas_call(
        _fwd_body,
        grid=(_G,),
        in_specs=in_specs,
        out_specs=pl.BlockSpec((1, 1, _NCLS), lambda g: (g, 0, 0)),
        out_shape=jax.ShapeDtypeStruct((_G, 1, _NCLS), jnp.float32),
    )(A3, x3, *weights)


def kernel(x, edge_index, batch, W1, b1, Ws1, bs1, W2, b2, Ws2, bs2,
           W3, b3, Ws3, bs3, Wl1, bl1, Wl2, bl2, Wl3, bl3):
    del batch
    A3 = _build_adj(edge_index).reshape(_G, _NPER, _PAD)
    x3 = x.reshape(_G, _NPER, 128)
    out = _forward_tc(
        A3, x3, W1, b1.reshape(1, -1), Ws1, bs1.reshape(1, -1),
        W2, b2.reshape(1, -1), Ws2, bs2.reshape(1, -1),
        W3, b3.reshape(1, -1), Ws3, bs3.reshape(1, -1),
        Wl1, bl1.reshape(1, -1), Wl2, bl2.reshape(1, -1),
        Wl3, bl3.reshape(1, -1))
    return out.reshape(_G, _NCLS)


# speedup vs baseline: 121.4108x; 1.4447x over previous
import functools

import jax
import jax.numpy as jnp
from jax import lax
from jax.experimental import pallas as pl
from jax.experimental.pallas import tpu as pltpu
from jax.experimental.pallas import tpu_sc as plsc

_N, _E, _G, _NPER = 10000, 320000, 50, 200
_PAD = 256
_NCLS = 10

_NSUB = 16
_EPT = _E // _NSUB
_ECH = 2000
_NCH = _EPT // _ECH
_ROWS = (_ECH + 127) // 128
_SLOT = _ROWS * 128
_HALF = _N // 2
_HALFE = _HALF * _PAD
_STRIPE = _HALFE // _NSUB
_ZCH = 2048
_NZ = _STRIPE // _ZCH


def _build_adj(edge_index):
    mesh = plsc.VectorSubcoreMesh(core_axis_name="c", subcore_axis_name="s")

    @functools.partial(
        pl.kernel,
        mesh=mesh,
        out_type=jax.ShapeDtypeStruct((_N * _PAD,), jnp.float32),
        scratch_types=[
            pltpu.VMEM((2 * _SLOT,), jnp.int32),
            pltpu.VMEM((2 * _SLOT,), jnp.int32),
            pltpu.VMEM((_NCH * _ROWS, 128), jnp.int32),
            pltpu.VMEM((128,), jnp.float32),
            pltpu.VMEM((_ZCH,), jnp.float32),
            pltpu.VMEM_SHARED((_HALFE,), jnp.float32),
            pltpu.SemaphoreType.DMA,
            pltpu.SemaphoreType.DMA,
            pltpu.SemaphoreType.DMA,
        ],
    )
    def build(src_hbm, dst_hbm, out_hbm, src_v, dst_v, idx_v, ones_v, z_v,
              shared, sem_z, sem_in, sem_sc):
        c = lax.axis_index("c")
        s = lax.axis_index("s")

        def fill_ones(i, _):
            ones_v[pl.ds(i * 16, 16)] = jnp.ones((16,), jnp.float32)
            return 0

        lax.fori_loop(0, 128 // 16, fill_ones, 0)

        def fill_z(i, _):
            z_v[pl.ds(i * 16, 16)] = jnp.zeros((16,), jnp.float32)
            return 0

        lax.fori_loop(0, _ZCH // 16, fill_z, 0)

        def zfire(q, _):
            pltpu.async_copy(
                z_v, shared.at[pl.ds(s * _STRIPE + q * _ZCH, _ZCH)], sem_z)
            return 0

        lax.fori_loop(0, _NZ, zfire, 0)

        pltpu.async_copy(src_hbm.at[pl.ds(s * _EPT, _ECH)],
                         src_v.at[pl.ds(0, _ECH)], sem_in)
        pltpu.async_copy(dst_hbm.at[pl.ds(s * _EPT, _ECH)],
                         dst_v.at[pl.ds(0, _ECH)], sem_in)

        def zdrain(q, _):
            pltpu.make_async_copy(
                z_v, shared.at[pl.ds(s * _STRIPE, _ZCH)], sem_z).wait()
            return 0

        lax.fori_loop(0, _NZ, zdrain, 0)
        plsc.subcore_barrier()
        lo = c * _HALF

        def chunk(q, _):
            p = lax.rem(q, 2)
            pltpu.make_async_copy(src_hbm.at[pl.ds(0, _ECH)],
                                  src_v.at[pl.ds(p * _SLOT, _ECH)], sem_in).wait()
            pltpu.make_async_copy(dst_hbm.at[pl.ds(0, _ECH)],
                                  dst_v.at[pl.ds(p * _SLOT, _ECH)], sem_in).wait()

            @pl.when(q + 1 < _NCH)
            def _():
                off = s * _EPT + (q + 1) * _ECH
                pltpu.async_copy(src_hbm.at[pl.ds(off, _ECH)],
                                 src_v.at[pl.ds((1 - p) * _SLOT, _ECH)], sem_in)
                pltpu.async_copy(dst_hbm.at[pl.ds(off, _ECH)],
                                 dst_v.at[pl.ds((1 - p) * _SLOT, _ECH)], sem_in)

            @pl.when(q >= 2)
            def _():
                def scdrain(i, _):
                    pltpu.make_async_copy(
                        ones_v, shared.at[idx_v.at[0]], sem_sc).wait()
                    return 0

                lax.fori_loop(0, _ROWS, scdrain, 0)

            def rowfn(j, _):
                for v in range(8):
                    base = j * 128 + v * 16
                    sv = src_v[pl.ds(p * _SLOT + base, 16)]
                    dv = dst_v[pl.ds(p * _SLOT + base, 16)]
                    col = lax.rem(sv, 200)
                    dl = dv - lo
                    lane = base + lax.iota(jnp.int32, 16)
                    ok = (dl >= 0) & (dl < _HALF) & (lane < _ECH)
                    idx_v[q * _ROWS + j, pl.ds(v * 16, 16)] = jnp.where(
                        ok, dl * _PAD + col, 255)
                pltpu.async_copy(
                    ones_v, shared.at[idx_v.at[q * _ROWS + j]], sem_sc,
                    add=True)
                return 0

            lax.fori_loop(0, _ROWS, rowfn, 0)
            return 0

        lax.fori_loop(0, _NCH, chunk, 0)

        def scdrain_tail(i, _):
            pltpu.make_async_copy(
                ones_v, shared.at[idx_v.at[0]], sem_sc).wait()
            return 0

        lax.fori_loop(0, 2 * _ROWS, scdrain_tail, 0)
        plsc.subcore_barrier()
        pltpu.sync_copy(shared.at[pl.ds(s * _STRIPE, _STRIPE)],
                        out_hbm.at[pl.ds(c * _HALFE + s * _STRIPE, _STRIPE)])

    return build(edge_index[0], edge_index[1])


def _dinv_t(A, active):
    ap = jnp.concatenate(
        [active, jnp.zeros((_PAD - _NPER, 1), jnp.float32)], axis=0)
    deg = jnp.dot(A, ap, preferred_element_type=jnp.float32) * active + active
    return jnp.where(deg > 0, 1.0 / jnp.sqrt(jnp.maximum(deg, 1e-12)), 0.0)


def _gcn_t(A, h, W, b, active, dinv):
    hw = jnp.dot(h, W, preferred_element_type=jnp.float32)
    hp = jnp.concatenate(
        [dinv * hw, jnp.zeros((_PAD - _NPER, hw.shape[1]), jnp.float32)], axis=0)
    agg = jnp.dot(A, hp, preferred_element_type=jnp.float32) * dinv
    return (agg + (dinv * dinv * active) * hw + b) * active


_GPB = 5


def _layer_multi(As, hs_in, acts, W, b, Ws, bs, k, eye, ilt):
    f32 = jnp.float32
    zpad1 = jnp.zeros((_PAD - _NPER, 1), f32)
    aps = [jnp.concatenate([a, zpad1], axis=0) for a in acts]
    degs = [jnp.dot(A, ap, preferred_element_type=f32)
            for A, ap in zip(As, aps)]
    degs = [d * a + a for d, a in zip(degs, acts)]
    dinvs = [jnp.where(d > 0, 1.0 / jnp.sqrt(jnp.maximum(d, 1e-12)), 0.0)
             for d in degs]
    hws = [jnp.dot(h, W, preferred_element_type=f32) for h in hs_in]
    zpadF = jnp.zeros((_PAD - _NPER, hws[0].shape[1]), f32)
    hps = [jnp.concatenate([di * hw, zpadF], axis=0)
           for di, hw in zip(dinvs, hws)]
    aggs = [jnp.dot(A, hp, preferred_element_type=f32) * di
            for A, hp, di in zip(As, hps, dinvs)]
    hs = [jax.nn.relu((ag + (di * di * a) * hw + b) * a)
          for ag, di, a, hw in zip(aggs, dinvs, acts, hws)]
    hss = [jnp.dot(h, Ws, preferred_element_type=f32) for h in hs]
    hpss = [jnp.concatenate([di * hw, zpad1], axis=0)
            for di, hw in zip(dinvs, hss)]
    aggss = [jnp.dot(A, hp, preferred_element_type=f32) * di
             for A, hp, di in zip(As, hpss, dinvs)]
    scs = [(ag + (di * di * a) * hw + bs) * a
           for ag, di, a, hw in zip(aggss, dinvs, acts, hss)]
    sms = [jnp.where(a > 0, s, -1e9) for s, a in zip(scs, acts)]
    smTs = [lax.dot_general(sm, eye, (((0,), (0,)), ((), ()))) for sm in sms]
    cnts = [jnp.sum(jnp.where((smT > sm) | ((smT == sm) & ilt), 1.0, 0.0),
                    axis=1, keepdims=True)
            for smT, sm in zip(smTs, sms)]
    nas = [jnp.where(c < k, 1.0, 0.0) * a for c, a in zip(cnts, acts)]
    hns = [h * jnp.tanh(s) * na for h, s, na in zip(hs, scs, nas)]
    xmaxs = [jnp.max(jnp.where(na > 0, hn, -1e9), axis=0, keepdims=True)
             for hn, na in zip(hns, nas)]
    cntns = [jnp.maximum(jnp.sum(na), 1.0) for na in nas]
    xmeans = [jnp.sum(hn, axis=0, keepdims=True) / cn
              for hn, cn in zip(hns, cntns)]
    Ts = [jnp.concatenate([xm, xme], axis=1)
          for xm, xme in zip(xmaxs, xmeans)]
    return hns, nas, Ts


def _fwd_body(a_ref, x_ref, W1r, b1r, Ws1r, bs1r, W2r, b2r, Ws2r, bs2r,
              W3r, b3r, Ws3r, bs3r, Wl1r, bl1r, Wl2r, bl2r, Wl3r, bl3r,
              o_ref):
    f32 = jnp.float32
    ii = lax.broadcasted_iota(jnp.int32, (_NPER, _NPER), 0)
    jj = lax.broadcasted_iota(jnp.int32, (_NPER, _NPER), 1)
    eye = jnp.where(ii == jj, 1.0, 0.0).astype(f32)
    ilt = jj < ii
    As = [a_ref[b] for b in range(_GPB)]
    hs = [x_ref[b] for b in range(_GPB)]
    acts = [jnp.ones((_NPER, 1), f32) for _ in range(_GPB)]
    hs, acts, T1s = _layer_multi(As, hs, acts, W1r[...], b1r[...], Ws1r[...],
                                 bs1r[...], 100, eye, ilt)
    hs, acts, T2s = _layer_multi(As, hs, acts, W2r[...], b2r[...], Ws2r[...],
                                 bs2r[...], 50, eye, ilt)
    hs, acts, T3s = _layer_multi(As, hs, acts, W3r[...], b3r[...], Ws3r[...],
                                 bs3r[...], 25, eye, ilt)
    zs = [jnp.concatenate([t1, t2, t3], axis=1)
          for t1, t2, t3 in zip(T1s, T2s, T3s)]
    zs = [jax.nn.relu(jnp.dot(z, Wl1r[...], preferred_element_type=f32)
                      + bl1r[...]) for z in zs]
    zs = [jax.nn.relu(jnp.dot(z, Wl2r[...], preferred_element_type=f32)
                      + bl2r[...]) for z in zs]
    zs = [jnp.dot(z, Wl3r[...], preferred_element_type=f32) + bl3r[...]
          for z in zs]
    for bi, z in enumerate(zs):
        m = jnp.max(z, axis=1, keepdims=True)
        o_ref[bi] = (z - m) - jnp.log(
            jnp.sum(jnp.exp(z - m), axis=1, keepdims=True))


def _forward_tc(A3, x3, *weights):
    def fullspec(arr):
        nd = arr.ndim
        return pl.BlockSpec(arr.shape, lambda g, _nd=nd: (0,) * _nd)

    in_specs = [
        pl.BlockSpec((_GPB, _NPER, _PAD), lambda g: (g, 0, 0)),
        pl.BlockSpec((_GPB, _NPER, 128), lambda g: (g, 0, 0)),
    ] + [fullspec(w) for w in weights]
    return pl.pallas_call(
        _fwd_body,
        grid=(_G // _GPB,),
        in_specs=in_specs,
        out_specs=pl.BlockSpec((_GPB, 1, _NCLS), lambda g: (g, 0, 0)),
        out_shape=jax.ShapeDtypeStruct((_G, 1, _NCLS), jnp.float32),
    )(A3, x3, *weights)


def kernel(x, edge_index, batch, W1, b1, Ws1, bs1, W2, b2, Ws2, bs2,
           W3, b3, Ws3, bs3, Wl1, bl1, Wl2, bl2, Wl3, bl3):
    del batch
    A3 = _build_adj(edge_index).reshape(_G, _NPER, _PAD)
    x3 = x.reshape(_G, _NPER, 128)
    out = _forward_tc(
        A3, x3, W1, b1.reshape(1, -1), Ws1, bs1.reshape(1, -1),
        W2, b2.reshape(1, -1), Ws2, bs2.reshape(1, -1),
        W3, b3.reshape(1, -1), Ws3, bs3.reshape(1, -1),
        Wl1, bl1.reshape(1, -1), Wl2, bl2.reshape(1, -1),
        Wl3, bl3.reshape(1, -1))
    return out.reshape(_G, _NCLS)
